# Initial kernel scaffold; baseline (speedup 1.0000x reference)
import jax
import jax.numpy as jnp
from jax.experimental import pallas as pl


def kernel(w, dat, labels):
    raise NotImplementedError("write your pallas kernel here")


# speedup vs baseline: 1.9411x; 1.9411x over previous
import jax
import jax.numpy as jnp
from jax.experimental import pallas as pl
from jax.experimental.pallas import tpu as pltpu

_MARGIN = 0.2
_N = 8192
_D = 128
_BLK = 512
_STEPS = _N // _BLK


def _scores_body(dat_ref, w_ref, lab_ref, a_ref, b_ref):
    s = jnp.dot(dat_ref[...], w_ref[...],
                preferred_element_type=jnp.float32,
                precision=jax.lax.Precision.HIGHEST)
    pos = lab_ref[...] == 1
    s = jnp.where(pos, s - _MARGIN, s)
    a_ref[...] = jnp.where(pos, s, jnp.inf)
    b_ref[...] = jnp.where(pos, -jnp.inf, s)


def _pairwise_body(a_ref, b_ref, out_ref, tot_ref, npos_ref):
    i = pl.program_id(0)

    @pl.when(i == 0)
    def _init():
        tot_ref[0] = 0.0
        npos_ref[0] = 0

    a_col = a_ref[...]
    b_row = b_ref[...]
    tile = jnp.maximum(b_row - a_col, 0.0)
    tot_ref[0] += jnp.sum(tile)
    npos_ref[0] += jnp.sum((a_col != jnp.inf).astype(jnp.int32))

    @pl.when(i == _STEPS - 1)
    def _finalize():
        npos = npos_ref[0]
        npairs = (npos * (_N - npos)).astype(jnp.float32)
        loss = jnp.where(npairs == 0.0, 0.0, tot_ref[0] / npairs)
        out_ref[...] = jnp.full((1, 1), loss, dtype=jnp.float32)


def kernel(w, dat, labels):
    n, d = dat.shape
    a, b = pl.pallas_call(
        _scores_body,
        out_shape=(
            jax.ShapeDtypeStruct((n, 1), jnp.float32),
            jax.ShapeDtypeStruct((n, 1), jnp.float32),
        ),
    )(dat, w.reshape(d, 1), labels.reshape(n, 1))

    out = pl.pallas_call(
        _pairwise_body,
        grid=(_STEPS,),
        in_specs=[
            pl.BlockSpec((_BLK, 1), lambda i: (i, 0)),
            pl.BlockSpec((1, n), lambda i: (0, 0)),
        ],
        out_specs=pl.BlockSpec((1, 1), lambda i: (0, 0)),
        out_shape=jax.ShapeDtypeStruct((1, 1), jnp.float32),
        scratch_shapes=[
            pltpu.SMEM((1,), jnp.float32),
            pltpu.SMEM((1,), jnp.int32),
        ],
    )(a, b.reshape(1, n))
    return out.reshape(())


# speedup vs baseline: 2.3931x; 1.2329x over previous
import functools

import jax
import jax.numpy as jnp
from jax import lax
from jax.experimental import pallas as pl
from jax.experimental.pallas import tpu as pltpu
from jax.experimental.pallas import tpu_sc as plsc

_MARGIN = 0.2
_N = 8192
_D = 128
_NC = 2
_NS = 16
_NW = _NC * _NS
_L = 16
_NV = _N // _L
_U = 8


def _scores_body(dat_ref, w_ref, lab_ref, a_ref, b_ref):
    s = jnp.dot(dat_ref[...], w_ref[...],
                preferred_element_type=jnp.float32,
                precision=lax.Precision.HIGHEST)
    pos = lab_ref[...] == 1
    s = jnp.where(pos, s - _MARGIN, s)
    a_ref[...] = jnp.where(pos, s, jnp.inf)
    b_ref[...] = jnp.where(pos, -jnp.inf, s)


_sc_mesh = plsc.VectorSubcoreMesh(core_axis_name="c", subcore_axis_name="s")


@functools.partial(
    pl.kernel,
    out_type=jax.ShapeDtypeStruct((_NW, _L), jnp.float32),
    mesh=_sc_mesh,
    compiler_params=pltpu.CompilerParams(needs_layout_passes=False),
    scratch_types=[
        pltpu.VMEM((_N,), jnp.float32),
        pltpu.VMEM((_N,), jnp.float32),
        pltpu.VMEM((_N,), jnp.float32),
        pltpu.VMEM((_N,), jnp.float32),
        pltpu.VMEM((_L,), jnp.float32),
    ],
)
def _sc_pairwise(a_hbm, b_hbm, out_hbm, a_v, b_v, pos_v, neg_v, acc_v):
    c = lax.axis_index("c")
    s = lax.axis_index("s")
    wid = s * _NC + c

    pltpu.sync_copy(a_hbm, a_v)
    pltpu.sync_copy(b_hbm, b_v)

    minf = jnp.full((_L,), -jnp.inf, dtype=jnp.float32)

    def fill_body(v, carry):
        neg_v[pl.ds(v * _L, _L)] = minf
        return carry

    lax.fori_loop(0, _NV, fill_body, 0)

    def compact_body(v, carry):
        cp, cn = carry
        av = a_v[pl.ds(v * _L, _L)]
        bv = b_v[pl.ds(v * _L, _L)]
        mp = av != jnp.inf
        mn = bv != -jnp.inf
        rp = plsc.cumsum(mp.astype(jnp.int32)) - 1
        rn = plsc.cumsum(mn.astype(jnp.int32)) - 1
        plsc.store_scatter(pos_v, [cp + rp], av, mask=mp)
        plsc.store_scatter(neg_v, [cn + rn], bv, mask=mn)
        cp = cp + plsc.all_reduce_population_count(mp)[0]
        cn = cn + plsc.all_reduce_population_count(mn)[0]
        return cp, cn

    npos, nneg = lax.fori_loop(0, _NV, compact_body, (0, 0))

    lo = (wid * npos) // _NW
    hi = ((wid + 1) * npos) // _NW
    nit = (nneg + _U * _L - 1) // (_U * _L)

    zeros = jnp.zeros((_L,), dtype=jnp.float32)

    def pos_body(k, acc):
        pvec = plsc.load_gather(pos_v, [jnp.full((_L,), k, dtype=jnp.int32)])

        def neg_body(v, accs):
            base = v * (_U * _L)
            out = []
            for u in range(_U):
                bvec = neg_v[pl.ds(base + u * _L, _L)]
                out.append(accs[u] + jnp.maximum(bvec - pvec, 0.0))
            return tuple(out)

        accs = lax.fori_loop(0, nit, neg_body, (acc,) + (zeros,) * (_U - 1))
        total = accs[0]
        for u in range(1, _U):
            total = total + accs[u]
        return total

    acc = lax.fori_loop(lo, hi, pos_body, zeros)
    acc_v[...] = acc
    pltpu.sync_copy(acc_v, out_hbm.at[wid])


def _finalize_body(part_ref, lab_ref, out_ref):
    total = jnp.sum(part_ref[...])
    npos = jnp.sum((lab_ref[...] == 1).astype(jnp.int32))
    npairs = (npos * (_N - npos)).astype(jnp.float32)
    loss = jnp.where(npairs == 0.0, 0.0, total / npairs)
    out_ref[...] = jnp.full((1, 1), loss, dtype=jnp.float32)


def kernel(w, dat, labels):
    n, d = dat.shape
    a, b = pl.pallas_call(
        _scores_body,
        out_shape=(
            jax.ShapeDtypeStruct((n, 1), jnp.float32),
            jax.ShapeDtypeStruct((n, 1), jnp.float32),
        ),
    )(dat, w.reshape(d, 1), labels.reshape(n, 1))

    partials = _sc_pairwise(a.reshape(n), b.reshape(n))

    out = pl.pallas_call(
        _finalize_body,
        out_shape=jax.ShapeDtypeStruct((1, 1), jnp.float32),
    )(partials, labels.reshape(n, 1))
    return out.reshape(())
